# baseline (speedup 1.0000x reference)
import jax
import jax.numpy as jnp
import numpy as np
from jax import lax
from jax.experimental import pallas as pl
from jax.experimental.pallas import tpu as pltpu
from jax.experimental.pallas import tpu_sc as plsc

VOCAB = 1000000
USER = 100000
D = 64
MAX_LEN = 200
B = 1024
L = 200

NC = 2
NS = 16
NW = NC * NS
SEQ_PER_W = B // NW
IDX_PER_W = SEQ_PER_W * L

CHUNKS = ((0, 128), (128, 72))


def _pe_table(max_len, d_model):
    pos = np.arange(max_len, dtype=np.float64)[:, None]
    div = np.exp(
        np.arange(0, d_model, 2, dtype=np.float64) * -(np.log(10000.0) / d_model)
    )
    pe = np.zeros((max_len, d_model), dtype=np.float32)
    pe[:, 0::2] = np.sin(pos * div).astype(np.float32)
    pe[:, 1::2] = np.cos(pos * div).astype(np.float32)
    return pe


_PE_FLAT = _pe_table(MAX_LEN, D).reshape(-1)


def _body(trow_hbm, toff_hbm, urow_hbm, uoff_hbm, tok2_hbm, usr2_hbm,
          pe_hbm, out_hbm,
          idx_all, hof_all, rows0, rows1, obuf0, obuf1, pe_v,
          urows_v, urow_v, uoff_v,
          gsem0, gsem1, osem0, osem1):
    wid = lax.axis_index("s") * NC + lax.axis_index("c")
    base = wid * SEQ_PER_W
    ibase = wid * IDX_PER_W

    pltpu.sync_copy(pe_hbm, pe_v)
    pltpu.sync_copy(trow_hbm.at[pl.ds(ibase, IDX_PER_W)], idx_all)
    pltpu.sync_copy(toff_hbm.at[pl.ds(ibase, IDX_PER_W)],
                    hof_all.at[pl.ds(0, IDX_PER_W)])
    pltpu.sync_copy(urow_hbm.at[pl.ds(base, SEQ_PER_W)], urow_v)
    pltpu.sync_copy(uoff_hbm.at[pl.ds(base, SEQ_PER_W)], uoff_v)
    pltpu.async_copy(usr2_hbm.at[urow_v], urows_v, gsem0).wait()

    rows = (rows0, rows1)
    obuf = (obuf0, obuf1)
    gsem = (gsem0, gsem1)
    osem = (osem0, osem1)

    def start_gather(j, p):
        for off, n in CHUNKS:
            pltpu.async_copy(
                tok2_hbm.at[idx_all.at[pl.ds(j * L + off, n)]],
                rows[p].at[pl.ds(off, n)],
                gsem[p],
            )

    def wait_gather(j, p):
        for off, n in CHUNKS:
            pltpu.make_async_copy(
                tok2_hbm.at[idx_all.at[pl.ds(j * L + off, n)]],
                rows[p].at[pl.ds(off, n)],
                gsem[p],
            ).wait()

    start_gather(0, 0)

    @pl.loop(0, SEQ_PER_W, step=2)
    def _(jj):
        for p in range(2):
            j = jj + p
            b = base + j
            q = 1 - p

            @pl.when(j + 1 < SEQ_PER_W)
            def _():
                start_gather(j + 1, q)

            wait_gather(j, p)

            @pl.when(j >= 2)
            def _():
                pltpu.make_async_copy(
                    obuf[p], out_hbm.at[pl.ds((b - 2) * L * D, L * D)], osem[p]
                ).wait()

            iota = lax.iota(jnp.int32, 16)
            jvec = jnp.full((16,), j, jnp.int32)
            uh = plsc.load_gather(uoff_v, [jvec])
            u = [
                plsc.load_gather(urows_v, [jvec, uh + (c * 16) + iota])
                for c in range(D // 16)
            ]
            rp = rows[p]
            op = obuf[p]

            def row_block(r0, nk):
                hof = hof_all[pl.ds(j * L + r0, 16)]
                for k in range(nk):
                    hf = hof[k]
                    r = r0 + k
                    for c in range(D // 16):
                        op[pl.ds(r * D + c * 16, 16)] = (
                            rp[r, pl.ds(hf + c * 16, 16)]
                            + pe_v[pl.ds(r * D + c * 16, 16)]
                            + u[c]
                        )

            @pl.loop(0, (L // 16) * 16, step=16)
            def _(r0):
                row_block(r0, 16)

            if L % 16:
                row_block((L // 16) * 16, L % 16)

            pltpu.async_copy(op, out_hbm.at[pl.ds(b * L * D, L * D)], osem[p])

    last = base + SEQ_PER_W
    pltpu.make_async_copy(
        obuf[0], out_hbm.at[pl.ds((last - 2) * L * D, L * D)], osem[0]
    ).wait()
    pltpu.make_async_copy(
        obuf[1], out_hbm.at[pl.ds((last - 1) * L * D, L * D)], osem[1]
    ).wait()


@jax.jit
def _run(trow, toff, urow, uoff, tok2, usr2, pe_flat):
    mesh = plsc.VectorSubcoreMesh(core_axis_name="c", subcore_axis_name="s")
    f = pl.kernel(
        _body,
        out_type=jax.ShapeDtypeStruct((B * L * D,), jnp.float32),
        mesh=mesh,
        scratch_types=[
            pltpu.VMEM((IDX_PER_W,), jnp.int32),
            pltpu.VMEM((IDX_PER_W + 16,), jnp.int32),
            pltpu.VMEM((L, 128), jnp.float32),
            pltpu.VMEM((L, 128), jnp.float32),
            pltpu.VMEM((L * D,), jnp.float32),
            pltpu.VMEM((L * D,), jnp.float32),
            pltpu.VMEM((L * D,), jnp.float32),
            pltpu.VMEM((SEQ_PER_W, 128), jnp.float32),
            pltpu.VMEM((SEQ_PER_W,), jnp.int32),
            pltpu.VMEM((SEQ_PER_W,), jnp.int32),
            pltpu.SemaphoreType.DMA,
            pltpu.SemaphoreType.DMA,
            pltpu.SemaphoreType.DMA,
            pltpu.SemaphoreType.DMA,
        ],
        compiler_params=pltpu.CompilerParams(needs_layout_passes=False),
    )
    return f(trow, toff, urow, uoff, tok2, usr2, pe_flat)


def kernel(sequence, user_idx, token_table, user_table):
    v = sequence.astype(jnp.int32)
    u = user_idx.astype(jnp.int32)
    trow = (v >> 1).reshape(-1)
    toff = ((v & 1) * D).reshape(-1)
    urow = u >> 1
    uoff = (u & 1) * D
    tok2 = jnp.concatenate([token_table[0::2], token_table[1::2]], axis=1)
    usr2 = jnp.concatenate([user_table[0::2], user_table[1::2]], axis=1)
    out_flat = _run(trow, toff, urow, uoff, tok2, usr2, _PE_FLAT)
    return out_flat.reshape(B, L, D)


# speedup vs baseline: 12.8256x; 12.8256x over previous
import jax
import jax.numpy as jnp
import numpy as np
from jax import lax
from jax.experimental import pallas as pl
from jax.experimental.pallas import tpu as pltpu
from jax.experimental.pallas import tpu_sc as plsc

VOCAB = 1000000
USER = 100000
D = 64
MAX_LEN = 200
B = 1024
L = 200

NC = 2
NS = 16
NW = NC * NS
SEQ_PER_W = B // NW
IDX_PER_W = SEQ_PER_W * L

CHUNKS = ((0, 128), (128, 72))


def _pe_table(max_len, d_model):
    pos = np.arange(max_len, dtype=np.float64)[:, None]
    div = np.exp(
        np.arange(0, d_model, 2, dtype=np.float64) * -(np.log(10000.0) / d_model)
    )
    pe = np.zeros((max_len, d_model), dtype=np.float32)
    pe[:, 0::2] = np.sin(pos * div).astype(np.float32)
    pe[:, 1::2] = np.cos(pos * div).astype(np.float32)
    return pe


_PE_FLAT = _pe_table(MAX_LEN, D).reshape(-1)


def _body(trow_hbm, urow_hbm, tok_hbm, usr_hbm, pe_hbm, out_hbm,
          idx_all, rows0, rows1, obuf0, obuf1, pe_v, urows_v, urow_v,
          gsem0, gsem1, osem0, osem1):
    wid = lax.axis_index("s") * NC + lax.axis_index("c")
    base = wid * SEQ_PER_W
    ibase = wid * IDX_PER_W

    pltpu.sync_copy(pe_hbm, pe_v)
    pltpu.sync_copy(trow_hbm.at[pl.ds(ibase, IDX_PER_W)], idx_all)
    pltpu.sync_copy(urow_hbm.at[pl.ds(base, SEQ_PER_W)], urow_v)
    pltpu.async_copy(usr_hbm.at[urow_v], urows_v, gsem0).wait()

    rows = (rows0, rows1)
    obuf = (obuf0, obuf1)
    gsem = (gsem0, gsem1)
    osem = (osem0, osem1)

    def start_gather(j, p):
        for off, n in CHUNKS:
            pltpu.async_copy(
                tok_hbm.at[idx_all.at[pl.ds(j * L + off, n)]],
                rows[p].at[pl.ds(off, n)],
                gsem[p],
            )

    def wait_gather(j, p):
        for off, n in CHUNKS:
            pltpu.make_async_copy(
                tok_hbm.at[idx_all.at[pl.ds(j * L + off, n)]],
                rows[p].at[pl.ds(off, n)],
                gsem[p],
            ).wait()

    start_gather(0, 0)

    @pl.loop(0, SEQ_PER_W, step=2)
    def _(jj):
        for p in range(2):
            j = jj + p
            b = base + j
            q = 1 - p

            @pl.when(j + 1 < SEQ_PER_W)
            def _():
                start_gather(j + 1, q)

            wait_gather(j, p)

            @pl.when(j >= 2)
            def _():
                pltpu.make_async_copy(
                    obuf[p], out_hbm.at[pl.ds((b - 2) * L * D, L * D)], osem[p]
                ).wait()

            u = [urows_v[j, pl.ds(c * 16, 16)] for c in range(D // 16)]
            rp = rows[p]
            op = obuf[p]

            @plsc.parallel_loop(0, L, unroll=4)
            def _(r):
                for c in range(D // 16):
                    op[pl.ds(r * D + c * 16, 16)] = (
                        rp[r, pl.ds(c * 16, 16)]
                        + pe_v[pl.ds(r * D + c * 16, 16)]
                        + u[c]
                    )

            pltpu.async_copy(op, out_hbm.at[pl.ds(b * L * D, L * D)], osem[p])

    last = base + SEQ_PER_W
    pltpu.make_async_copy(
        obuf[0], out_hbm.at[pl.ds((last - 2) * L * D, L * D)], osem[0]
    ).wait()
    pltpu.make_async_copy(
        obuf[1], out_hbm.at[pl.ds((last - 1) * L * D, L * D)], osem[1]
    ).wait()


@jax.jit
def _run(trow, urow, tok_pad, usr_pad, pe_flat):
    mesh = plsc.VectorSubcoreMesh(core_axis_name="c", subcore_axis_name="s")
    f = pl.kernel(
        _body,
        out_type=jax.ShapeDtypeStruct((B * L * D,), jnp.float32),
        mesh=mesh,
        scratch_types=[
            pltpu.VMEM((IDX_PER_W,), jnp.int32),
            pltpu.VMEM((L, 128), jnp.float32),
            pltpu.VMEM((L, 128), jnp.float32),
            pltpu.VMEM((L * D,), jnp.float32),
            pltpu.VMEM((L * D,), jnp.float32),
            pltpu.VMEM((L * D,), jnp.float32),
            pltpu.VMEM((SEQ_PER_W, 128), jnp.float32),
            pltpu.VMEM((SEQ_PER_W,), jnp.int32),
            pltpu.SemaphoreType.DMA,
            pltpu.SemaphoreType.DMA,
            pltpu.SemaphoreType.DMA,
            pltpu.SemaphoreType.DMA,
        ],
    )
    return f(trow, urow, tok_pad, usr_pad, pe_flat)


def kernel(sequence, user_idx, token_table, user_table):
    trow = sequence.astype(jnp.int32).reshape(-1)
    urow = user_idx.astype(jnp.int32)
    tok_pad = jnp.pad(token_table, ((0, 0), (0, 128 - D)))
    usr_pad = jnp.pad(user_table, ((0, 0), (0, 128 - D)))
    out_flat = _run(trow, urow, tok_pad, usr_pad, _PE_FLAT)
    return out_flat.reshape(B, L, D)
